# baseline (speedup 1.0000x reference)
import functools

import jax
import jax.numpy as jnp
from jax import lax
from jax.experimental import pallas as pl
from jax.experimental.pallas import tpu as pltpu
from jax.experimental.pallas import tpu_sc as plsc

N_NODES = 10000
N_PAD = 10240
DIM = 128
N_EDGES = 320000
CHUNK = 128
NSC = 2
NTILES = 16
NBUF = 4
G = 16
N_CHUNKS = -(-N_EDGES // (NTILES * CHUNK * G)) * G
EDGES_PER_TILE = N_CHUNKS * CHUNK
E_PAD = EDGES_PER_TILE * NTILES
N_GROUPS = N_CHUNKS // NBUF
NGRP = N_CHUNKS // G
ROWS_PER_TILE = N_PAD // NTILES
SRC_FILL = 10100
DST_FILL = 10200

_mesh = plsc.VectorSubcoreMesh(core_axis_name="c", subcore_axis_name="s")


@functools.partial(
    pl.kernel,
    mesh=_mesh,
    out_type=jax.ShapeDtypeStruct((NSC, N_PAD), jnp.float32),
    scratch_types=[
        pltpu.VMEM((N_CHUNKS, CHUNK), jnp.int32),
        pltpu.VMEM((CHUNK,), jnp.float32),
        pltpu.VMEM((ROWS_PER_TILE,), jnp.float32),
        pltpu.VMEM_SHARED((N_PAD,), jnp.float32),
        pltpu.SemaphoreType.DMA,
    ],
)
def _deg_kernel(dst_hbm, deg_hbm, dst_v, ones_v, zero_v, acc_sh, sem):
    c = lax.axis_index("c")
    s = lax.axis_index("s")

    pltpu.sync_copy(dst_hbm.at[c, s], dst_v)

    def fill1(i, _):
        ones_v[pl.ds(i * 16, 16)] = jnp.ones((16,), jnp.float32)
        return 0
    lax.fori_loop(0, CHUNK // 16, fill1, 0)

    def fill0(i, _):
        zero_v[pl.ds(i * 16, 16)] = jnp.zeros((16,), jnp.float32)
        return 0
    lax.fori_loop(0, ROWS_PER_TILE // 16, fill0, 0)

    pltpu.sync_copy(zero_v, acc_sh.at[pl.ds(s * ROWS_PER_TILE, ROWS_PER_TILE)])
    plsc.subcore_barrier()

    def body(g, _):
        for b in range(NBUF):
            pltpu.async_copy(ones_v, acc_sh.at[dst_v.at[g * NBUF + b]], sem,
                             add=True)
        for b in range(NBUF):
            pltpu.make_async_copy(ones_v, acc_sh.at[dst_v.at[0]], sem).wait()
        return 0
    lax.fori_loop(0, N_GROUPS, body, 0)

    plsc.subcore_barrier()
    pltpu.sync_copy(acc_sh.at[pl.ds(s * ROWS_PER_TILE, ROWS_PER_TILE)],
                    deg_hbm.at[c, pl.ds(s * ROWS_PER_TILE, ROWS_PER_TILE)])


CH = 4096
N_CH = E_PAD // CH
GRP = CH // 16
NPASS = 2
COLS = 4
NBLK_COL = DIM // COLS


def _agg_body(table_hbm, srcg_hbm, dstg_hbm, zeros_hbm, out_hbm,
              tab_v, acc_v, sbuf, dbuf, isem):
    c = lax.axis_index("c")
    s = lax.axis_index("s")

    for P in range(NPASS):
        blk = P * NTILES + s
        pltpu.sync_copy(table_hbm.at[c, blk], tab_v)
        pltpu.sync_copy(zeros_hbm, acc_v)
        pltpu.sync_copy(srcg_hbm.at[c, 0], sbuf.at[0])
        pltpu.sync_copy(dstg_hbm.at[c, 0], dbuf.at[0])

        def chunk(ch, _):
            q = lax.rem(ch, 2)

            @pl.when(ch < N_CH - 1)
            def _():
                pltpu.async_copy(srcg_hbm.at[c, ch + 1], sbuf.at[1 - q], isem)
                pltpu.async_copy(dstg_hbm.at[c, ch + 1], dbuf.at[1 - q], isem)

            def grp_body(g, _):
                sidx = sbuf[q, pl.ds(g * 16, 16)] * COLS
                didx = dbuf[q, pl.ds(g * 16, 16)] * COLS
                for j in range(COLS):
                    vals = plsc.load_gather(tab_v, [sidx + j])
                    plsc.addupdate_scatter(acc_v, [didx + j], vals)
                return 0
            lax.fori_loop(0, GRP, grp_body, 0)

            @pl.when(ch < N_CH - 1)
            def _():
                pltpu.make_async_copy(srcg_hbm.at[c, 0], sbuf.at[1 - q],
                                      isem).wait()
                pltpu.make_async_copy(dstg_hbm.at[c, 0], dbuf.at[1 - q],
                                      isem).wait()
            return 0
        lax.fori_loop(0, N_CH, chunk, 0)

        pltpu.sync_copy(acc_v, out_hbm.at[c, blk])


_agg_kernel = functools.partial(
    pl.kernel,
    mesh=_mesh,
    out_type=jax.ShapeDtypeStruct((NSC, NBLK_COL, N_PAD * COLS), jnp.float32),
    scratch_types=[
        pltpu.VMEM((N_PAD * COLS,), jnp.float32),
        pltpu.VMEM((N_PAD * COLS,), jnp.float32),
        pltpu.VMEM((2, GRP * 16), jnp.int32),
        pltpu.VMEM((2, GRP * 16), jnp.int32),
        pltpu.SemaphoreType.DMA,
    ],
    compiler_params=pltpu.CompilerParams(needs_layout_passes=False),
)(_agg_body)


_BLK = 256
_NBLK = NSC * N_PAD // _BLK
_BLK_PER_G = N_PAD // _BLK


def _row_spec():
    return pl.BlockSpec((_BLK, DIM), lambda i: (i, 0))


def _deg_spec():
    return pl.BlockSpec((_BLK, 1), lambda i: (i, 0))


def _full_spec():
    return pl.BlockSpec((DIM, DIM), lambda i: (0, 0))


def _tc_matmul_scale(x_ref, w_ref, deg_ref, out_ref):
    out_ref[...] = jnp.dot(x_ref[...], w_ref[...],
                           preferred_element_type=jnp.float32) * lax.rsqrt(
                               deg_ref[...] + 1.0)


def _row_mask():
    i = pl.program_id(0)
    g = i // _BLK_PER_G
    base = i * _BLK - g * N_PAD
    rows = base + lax.broadcasted_iota(jnp.int32, (_BLK, 1), 0)
    return rows < N_NODES


def _tc_layer1(s_ref, h_ref, deg_ref, w_ref, b_ref, out_ref):
    inv = lax.rsqrt(deg_ref[...] + 1.0)
    h1 = jnp.maximum((s_ref[...] + h_ref[...]) * inv + b_ref[...], 0.0)
    h1s = jnp.dot(h1, w_ref[...], preferred_element_type=jnp.float32) * inv
    out_ref[...] = jnp.where(_row_mask(), h1s, 0.0)


def _tc_stats(s_ref, h_ref, deg_ref, b_ref, sum_ref, sq_ref):
    i = pl.program_id(0)
    inv = lax.rsqrt(deg_ref[...] + 1.0)
    agg = (s_ref[...] + h_ref[...]) * inv + b_ref[...]
    agg = jnp.where(_row_mask(), agg, 0.0)

    @pl.when(i % _BLK_PER_G == 0)
    def _():
        sum_ref[...] = jnp.zeros_like(sum_ref)
        sq_ref[...] = jnp.zeros_like(sq_ref)

    sum_ref[...] += jnp.sum(agg, axis=0)[None, None, :]
    sq_ref[...] += jnp.sum(agg * agg, axis=0)[None, None, :]


def _tc_standardize(s_ref, h_ref, deg_ref, b_ref, sum_ref, sq_ref, out_ref):
    inv = lax.rsqrt(deg_ref[...] + 1.0)
    agg = (s_ref[...] + h_ref[...]) * inv + b_ref[...]
    n = jnp.float32(N_NODES)
    mean = sum_ref[0] / n
    var = (sq_ref[0] - n * mean * mean) / (n - 1.0)
    out_ref[...] = (agg - mean) * lax.rsqrt(var)


def kernel(x1, edge_index1, x2, edge_index2, W0, b0, W1, b1):
    f32 = jnp.float32
    pad_n = N_PAD - N_NODES
    x_cat = jnp.concatenate([
        x1, jnp.zeros((pad_n, DIM), f32),
        x2, jnp.zeros((pad_n, DIM), f32)], axis=0)

    pad_e = E_PAD - N_EDGES
    def prep(ei):
        src = jnp.concatenate(
            [ei[0], jnp.full((pad_e,), SRC_FILL, jnp.int32)])
        dst = jnp.concatenate(
            [ei[1], jnp.full((pad_e,), DST_FILL, jnp.int32)])
        return src, dst
    s1, d1 = prep(edge_index1)
    s2, d2 = prep(edge_index2)
    dst_cat = jnp.stack([d1, d2]).reshape(NSC, NTILES, N_CHUNKS, CHUNK)
    srcg = jnp.stack([s1, s2]).reshape(NSC, N_CH, GRP * 16)
    dstg = jnp.stack([d1, d2]).reshape(NSC, N_CH, GRP * 16)
    zeros41 = jnp.zeros((N_PAD * COLS,), f32)

    deg = _deg_kernel(dst_cat).reshape(NSC * N_PAD, 1)

    b0r = b0.reshape(1, DIM)
    b1r = b1.reshape(1, DIM)
    grid = (_NBLK,)

    h0s = pl.pallas_call(
        _tc_matmul_scale,
        grid=grid,
        in_specs=[_row_spec(), _full_spec(), _deg_spec()],
        out_specs=_row_spec(),
        out_shape=jax.ShapeDtypeStruct((NSC * N_PAD, DIM), f32),
    )(x_cat, W0, deg)

    t0 = h0s.reshape(NSC, N_PAD, NBLK_COL, COLS).transpose(
        0, 2, 1, 3).reshape(NSC, NBLK_COL, N_PAD * COLS)
    s0 = _agg_kernel(t0, srcg, dstg, zeros41).reshape(
        NSC, NBLK_COL, N_PAD, COLS).transpose(0, 2, 1, 3).reshape(
        NSC * N_PAD, DIM)

    bias_spec = pl.BlockSpec((1, DIM), lambda i: (0, 0))
    h1s = pl.pallas_call(
        _tc_layer1,
        grid=grid,
        in_specs=[_row_spec(), _row_spec(), _deg_spec(), _full_spec(), bias_spec],
        out_specs=_row_spec(),
        out_shape=jax.ShapeDtypeStruct((NSC * N_PAD, DIM), f32),
    )(s0, h0s, deg, W1, b0r)

    t1 = h1s.reshape(NSC, N_PAD, NBLK_COL, COLS).transpose(
        0, 2, 1, 3).reshape(NSC, NBLK_COL, N_PAD * COLS)
    s1agg = _agg_kernel(t1, srcg, dstg, zeros41).reshape(
        NSC, NBLK_COL, N_PAD, COLS).transpose(0, 2, 1, 3).reshape(
        NSC * N_PAD, DIM)

    stat_spec = pl.BlockSpec((1, 1, DIM), lambda i: (i // _BLK_PER_G, 0, 0))
    colsum, colsq = pl.pallas_call(
        _tc_stats,
        grid=grid,
        in_specs=[_row_spec(), _row_spec(), _deg_spec(), bias_spec],
        out_specs=[stat_spec, stat_spec],
        out_shape=[jax.ShapeDtypeStruct((NSC, 1, DIM), f32)] * 2,
    )(s1agg, h1s, deg, b1r)

    z = pl.pallas_call(
        _tc_standardize,
        grid=grid,
        in_specs=[_row_spec(), _row_spec(), _deg_spec(), bias_spec,
                  stat_spec, stat_spec],
        out_specs=_row_spec(),
        out_shape=jax.ShapeDtypeStruct((NSC * N_PAD, DIM), f32),
    )(s1agg, h1s, deg, b1r, colsum, colsq)

    return z[:N_NODES], z[N_PAD:N_PAD + N_NODES]


# speedup vs baseline: 1.7407x; 1.7407x over previous
import functools

import jax
import jax.numpy as jnp
from jax import lax
from jax.experimental import pallas as pl
from jax.experimental.pallas import tpu as pltpu
from jax.experimental.pallas import tpu_sc as plsc

N_NODES = 10000
N_PAD = 10240
DIM = 128
N_EDGES = 320000
CHUNK = 128
NSC = 2
NTILES = 16
NBUF = 4
G = 16
N_CHUNKS = -(-N_EDGES // (NTILES * CHUNK * G)) * G
EDGES_PER_TILE = N_CHUNKS * CHUNK
E_PAD = EDGES_PER_TILE * NTILES
N_GROUPS = N_CHUNKS // NBUF
NGRP = N_CHUNKS // G
ROWS_PER_TILE = N_PAD // NTILES
SRC_FILL = 10100
DST_FILL = 10200

_mesh = plsc.VectorSubcoreMesh(core_axis_name="c", subcore_axis_name="s")


@functools.partial(
    pl.kernel,
    mesh=_mesh,
    out_type=jax.ShapeDtypeStruct((NSC, N_PAD), jnp.float32),
    scratch_types=[
        pltpu.VMEM((N_CHUNKS, CHUNK), jnp.int32),
        pltpu.VMEM((CHUNK,), jnp.float32),
        pltpu.VMEM((ROWS_PER_TILE,), jnp.float32),
        pltpu.VMEM_SHARED((N_PAD,), jnp.float32),
        pltpu.SemaphoreType.DMA,
    ],
)
def _deg_kernel(dst_hbm, deg_hbm, dst_v, ones_v, zero_v, acc_sh, sem):
    c = lax.axis_index("c")
    s = lax.axis_index("s")

    pltpu.sync_copy(dst_hbm.at[c, s], dst_v)

    def fill1(i, _):
        ones_v[pl.ds(i * 16, 16)] = jnp.ones((16,), jnp.float32)
        return 0
    lax.fori_loop(0, CHUNK // 16, fill1, 0)

    def fill0(i, _):
        zero_v[pl.ds(i * 16, 16)] = jnp.zeros((16,), jnp.float32)
        return 0
    lax.fori_loop(0, ROWS_PER_TILE // 16, fill0, 0)

    pltpu.sync_copy(zero_v, acc_sh.at[pl.ds(s * ROWS_PER_TILE, ROWS_PER_TILE)])
    plsc.subcore_barrier()

    def body(g, _):
        for b in range(NBUF):
            pltpu.async_copy(ones_v, acc_sh.at[dst_v.at[g * NBUF + b]], sem,
                             add=True)
        for b in range(NBUF):
            pltpu.make_async_copy(ones_v, acc_sh.at[dst_v.at[0]], sem).wait()
        return 0
    lax.fori_loop(0, N_GROUPS, body, 0)

    plsc.subcore_barrier()
    pltpu.sync_copy(acc_sh.at[pl.ds(s * ROWS_PER_TILE, ROWS_PER_TILE)],
                    deg_hbm.at[c, pl.ds(s * ROWS_PER_TILE, ROWS_PER_TILE)])


CH = 4096
N_CH = E_PAD // CH
GRP = CH // 16
NPASS = 2
COLS = 4
NBLK_COL = DIM // COLS


def _agg_body(table_hbm, srcg_hbm, dstg_hbm, zeros_hbm, out_hbm,
              tab_v, acc_v, sbuf, dbuf, isem):
    c = lax.axis_index("c")
    s = lax.axis_index("s")

    for P in range(NPASS):
        blk = P * NTILES + s
        pltpu.sync_copy(table_hbm.at[c, blk], tab_v)
        pltpu.sync_copy(zeros_hbm, acc_v)
        pltpu.sync_copy(srcg_hbm.at[c, 0], sbuf.at[0])
        pltpu.sync_copy(dstg_hbm.at[c, 0], dbuf.at[0])

        def chunk(ch, _):
            q = lax.rem(ch, 2)

            @pl.when(ch < N_CH - 1)
            def _():
                pltpu.async_copy(srcg_hbm.at[c, ch + 1], sbuf.at[1 - q], isem)
                pltpu.async_copy(dstg_hbm.at[c, ch + 1], dbuf.at[1 - q], isem)

            @plsc.parallel_loop(0, GRP * 16, step=16, unroll=8)
            def _(g16):
                sidx = sbuf[q, pl.ds(g16, 16)] * COLS
                didx = dbuf[q, pl.ds(g16, 16)] * COLS
                for j in range(COLS):
                    vals = plsc.load_gather(tab_v, [sidx + j])
                    plsc.addupdate_scatter(acc_v, [didx + j], vals)

            @pl.when(ch < N_CH - 1)
            def _():
                pltpu.make_async_copy(srcg_hbm.at[c, 0], sbuf.at[1 - q],
                                      isem).wait()
                pltpu.make_async_copy(dstg_hbm.at[c, 0], dbuf.at[1 - q],
                                      isem).wait()
            return 0
        lax.fori_loop(0, N_CH, chunk, 0)

        pltpu.sync_copy(acc_v, out_hbm.at[c, blk])


_agg_kernel = functools.partial(
    pl.kernel,
    mesh=_mesh,
    out_type=jax.ShapeDtypeStruct((NSC, NBLK_COL, N_PAD * COLS), jnp.float32),
    scratch_types=[
        pltpu.VMEM((N_PAD * COLS,), jnp.float32),
        pltpu.VMEM((N_PAD * COLS,), jnp.float32),
        pltpu.VMEM((2, GRP * 16), jnp.int32),
        pltpu.VMEM((2, GRP * 16), jnp.int32),
        pltpu.SemaphoreType.DMA,
    ],
    compiler_params=pltpu.CompilerParams(needs_layout_passes=False),
)(_agg_body)


_BLK = 256
_NBLK = NSC * N_PAD // _BLK
_BLK_PER_G = N_PAD // _BLK


def _row_spec():
    return pl.BlockSpec((_BLK, DIM), lambda i: (i, 0))


def _deg_spec():
    return pl.BlockSpec((_BLK, 1), lambda i: (i, 0))


def _full_spec():
    return pl.BlockSpec((DIM, DIM), lambda i: (0, 0))


def _tc_matmul_scale(x_ref, w_ref, deg_ref, out_ref):
    out_ref[...] = jnp.dot(x_ref[...], w_ref[...],
                           preferred_element_type=jnp.float32) * lax.rsqrt(
                               deg_ref[...] + 1.0)


def _row_mask():
    i = pl.program_id(0)
    g = i // _BLK_PER_G
    base = i * _BLK - g * N_PAD
    rows = base + lax.broadcasted_iota(jnp.int32, (_BLK, 1), 0)
    return rows < N_NODES


def _tc_layer1(s_ref, h_ref, deg_ref, w_ref, b_ref, out_ref):
    inv = lax.rsqrt(deg_ref[...] + 1.0)
    h1 = jnp.maximum((s_ref[...] + h_ref[...]) * inv + b_ref[...], 0.0)
    h1s = jnp.dot(h1, w_ref[...], preferred_element_type=jnp.float32) * inv
    out_ref[...] = jnp.where(_row_mask(), h1s, 0.0)


def _tc_stats(s_ref, h_ref, deg_ref, b_ref, sum_ref, sq_ref):
    i = pl.program_id(0)
    inv = lax.rsqrt(deg_ref[...] + 1.0)
    agg = (s_ref[...] + h_ref[...]) * inv + b_ref[...]
    agg = jnp.where(_row_mask(), agg, 0.0)

    @pl.when(i % _BLK_PER_G == 0)
    def _():
        sum_ref[...] = jnp.zeros_like(sum_ref)
        sq_ref[...] = jnp.zeros_like(sq_ref)

    sum_ref[...] += jnp.sum(agg, axis=0)[None, None, :]
    sq_ref[...] += jnp.sum(agg * agg, axis=0)[None, None, :]


def _tc_standardize(s_ref, h_ref, deg_ref, b_ref, sum_ref, sq_ref, out_ref):
    inv = lax.rsqrt(deg_ref[...] + 1.0)
    agg = (s_ref[...] + h_ref[...]) * inv + b_ref[...]
    n = jnp.float32(N_NODES)
    mean = sum_ref[0] / n
    var = (sq_ref[0] - n * mean * mean) / (n - 1.0)
    out_ref[...] = (agg - mean) * lax.rsqrt(var)


def kernel(x1, edge_index1, x2, edge_index2, W0, b0, W1, b1):
    f32 = jnp.float32
    pad_n = N_PAD - N_NODES
    x_cat = jnp.concatenate([
        x1, jnp.zeros((pad_n, DIM), f32),
        x2, jnp.zeros((pad_n, DIM), f32)], axis=0)

    pad_e = E_PAD - N_EDGES
    def prep(ei):
        src = jnp.concatenate(
            [ei[0], jnp.full((pad_e,), SRC_FILL, jnp.int32)])
        dst = jnp.concatenate(
            [ei[1], jnp.full((pad_e,), DST_FILL, jnp.int32)])
        return src, dst
    s1, d1 = prep(edge_index1)
    s2, d2 = prep(edge_index2)
    dst_cat = jnp.stack([d1, d2]).reshape(NSC, NTILES, N_CHUNKS, CHUNK)
    srcg = jnp.stack([s1, s2]).reshape(NSC, N_CH, GRP * 16)
    dstg = jnp.stack([d1, d2]).reshape(NSC, N_CH, GRP * 16)
    zeros41 = jnp.zeros((N_PAD * COLS,), f32)

    deg = _deg_kernel(dst_cat).reshape(NSC * N_PAD, 1)

    b0r = b0.reshape(1, DIM)
    b1r = b1.reshape(1, DIM)
    grid = (_NBLK,)

    h0s = pl.pallas_call(
        _tc_matmul_scale,
        grid=grid,
        in_specs=[_row_spec(), _full_spec(), _deg_spec()],
        out_specs=_row_spec(),
        out_shape=jax.ShapeDtypeStruct((NSC * N_PAD, DIM), f32),
    )(x_cat, W0, deg)

    t0 = h0s.reshape(NSC, N_PAD, NBLK_COL, COLS).transpose(
        0, 2, 1, 3).reshape(NSC, NBLK_COL, N_PAD * COLS)
    s0 = _agg_kernel(t0, srcg, dstg, zeros41).reshape(
        NSC, NBLK_COL, N_PAD, COLS).transpose(0, 2, 1, 3).reshape(
        NSC * N_PAD, DIM)

    bias_spec = pl.BlockSpec((1, DIM), lambda i: (0, 0))
    h1s = pl.pallas_call(
        _tc_layer1,
        grid=grid,
        in_specs=[_row_spec(), _row_spec(), _deg_spec(), _full_spec(), bias_spec],
        out_specs=_row_spec(),
        out_shape=jax.ShapeDtypeStruct((NSC * N_PAD, DIM), f32),
    )(s0, h0s, deg, W1, b0r)

    t1 = h1s.reshape(NSC, N_PAD, NBLK_COL, COLS).transpose(
        0, 2, 1, 3).reshape(NSC, NBLK_COL, N_PAD * COLS)
    s1agg = _agg_kernel(t1, srcg, dstg, zeros41).reshape(
        NSC, NBLK_COL, N_PAD, COLS).transpose(0, 2, 1, 3).reshape(
        NSC * N_PAD, DIM)

    stat_spec = pl.BlockSpec((1, 1, DIM), lambda i: (i // _BLK_PER_G, 0, 0))
    colsum, colsq = pl.pallas_call(
        _tc_stats,
        grid=grid,
        in_specs=[_row_spec(), _row_spec(), _deg_spec(), bias_spec],
        out_specs=[stat_spec, stat_spec],
        out_shape=[jax.ShapeDtypeStruct((NSC, 1, DIM), f32)] * 2,
    )(s1agg, h1s, deg, b1r)

    z = pl.pallas_call(
        _tc_standardize,
        grid=grid,
        in_specs=[_row_spec(), _row_spec(), _deg_spec(), bias_spec,
                  stat_spec, stat_spec],
        out_specs=_row_spec(),
        out_shape=jax.ShapeDtypeStruct((NSC * N_PAD, DIM), f32),
    )(s1agg, h1s, deg, b1r, colsum, colsq)

    return z[:N_NODES], z[N_PAD:N_PAD + N_NODES]


# speedup vs baseline: 2.5392x; 1.4587x over previous
import functools

import jax
import jax.numpy as jnp
from jax import lax
from jax.experimental import pallas as pl
from jax.experimental.pallas import tpu as pltpu
from jax.experimental.pallas import tpu_sc as plsc

N_NODES = 10000
N_PAD = 10240
DIM = 128
N_EDGES = 320000
CHUNK = 128
NSC = 2
NTILES = 16
NBUF = 4
G = 16
N_CHUNKS = -(-N_EDGES // (NTILES * CHUNK * G)) * G
EDGES_PER_TILE = N_CHUNKS * CHUNK
E_PAD = EDGES_PER_TILE * NTILES
N_GROUPS = N_CHUNKS // NBUF
ACH = 64
A_CHUNKS = EDGES_PER_TILE // ACH
AG = 32
ANGRP = A_CHUNKS // AG
ROWS_PER_TILE = N_PAD // NTILES
SRC_FILL = 10100
DST_FILL = 10200

_mesh = plsc.VectorSubcoreMesh(core_axis_name="c", subcore_axis_name="s")


@functools.partial(
    pl.kernel,
    mesh=_mesh,
    out_type=jax.ShapeDtypeStruct((NSC, N_PAD), jnp.float32),
    scratch_types=[
        pltpu.VMEM((N_CHUNKS, CHUNK), jnp.int32),
        pltpu.VMEM((CHUNK,), jnp.float32),
        pltpu.VMEM((ROWS_PER_TILE,), jnp.float32),
        pltpu.VMEM_SHARED((N_PAD,), jnp.float32),
        pltpu.SemaphoreType.DMA,
    ],
)
def _deg_kernel(dst_hbm, deg_hbm, dst_v, ones_v, zero_v, acc_sh, sem):
    c = lax.axis_index("c")
    s = lax.axis_index("s")

    pltpu.sync_copy(dst_hbm.at[c, s], dst_v)

    def fill1(i, _):
        ones_v[pl.ds(i * 16, 16)] = jnp.ones((16,), jnp.float32)
        return 0
    lax.fori_loop(0, CHUNK // 16, fill1, 0)

    def fill0(i, _):
        zero_v[pl.ds(i * 16, 16)] = jnp.zeros((16,), jnp.float32)
        return 0
    lax.fori_loop(0, ROWS_PER_TILE // 16, fill0, 0)

    pltpu.sync_copy(zero_v, acc_sh.at[pl.ds(s * ROWS_PER_TILE, ROWS_PER_TILE)])
    plsc.subcore_barrier()

    def body(g, _):
        for b in range(NBUF):
            pltpu.async_copy(ones_v, acc_sh.at[dst_v.at[g * NBUF + b]], sem,
                             add=True)
        for b in range(NBUF):
            pltpu.make_async_copy(ones_v, acc_sh.at[dst_v.at[0]], sem).wait()
        return 0
    lax.fori_loop(0, N_GROUPS, body, 0)

    plsc.subcore_barrier()
    pltpu.sync_copy(acc_sh.at[pl.ds(s * ROWS_PER_TILE, ROWS_PER_TILE)],
                    deg_hbm.at[c, pl.ds(s * ROWS_PER_TILE, ROWS_PER_TILE)])


def _agg_body(table_hbm, src_hbm, dst_hbm, out_hbm,
              sbuf, dbuf, rows_v, g0, g1, g2, g3, s0, s1, s2, s3,
              isem, acc_sh):
    c = lax.axis_index("c")
    s = lax.axis_index("s")
    gsems = [g0, g1, g2, g3]
    ssems = [s0, s1, s2, s3]

    def zrow(i, _):
        for j in range(DIM // 16):
            rows_v[0, i, pl.ds(j * 16, 16)] = jnp.zeros((16,), jnp.float32)
            rows_v[1, i, pl.ds(j * 16, 16)] = jnp.zeros((16,), jnp.float32)
        return 0
    lax.fori_loop(0, ACH, zrow, 0)
    for j in range(ROWS_PER_TILE // (2 * ACH)):
        pltpu.sync_copy(
            rows_v.at[0],
            acc_sh.at[pl.ds(s * ROWS_PER_TILE + 2 * j * ACH, ACH), :])
        pltpu.sync_copy(
            rows_v.at[1],
            acc_sh.at[pl.ds(s * ROWS_PER_TILE + (2 * j + 1) * ACH, ACH), :])
    plsc.subcore_barrier()

    pltpu.sync_copy(src_hbm.at[c, s, pl.ds(0, AG), :], sbuf.at[0])
    pltpu.sync_copy(dst_hbm.at[c, s, pl.ds(0, AG), :], dbuf.at[0])
    pltpu.async_copy(table_hbm.at[sbuf.at[0].at[0]], rows_v.at[0], g0)
    pltpu.async_copy(table_hbm.at[sbuf.at[0].at[1]], rows_v.at[1], g1)

    def grp(g, _):
        p = lax.rem(g, 2)

        for j in range(AG):
            b, b2 = j % NBUF, (j + 2) % NBUF
            pltpu.make_async_copy(table_hbm.at[sbuf.at[p].at[0]],
                                  rows_v.at[b], gsems[b]).wait()
            pltpu.async_copy(rows_v.at[b], acc_sh.at[dbuf.at[p].at[j]],
                             ssems[b], add=True)
            if j == 0:
                @pl.when(g > 0)
                def _():
                    pltpu.make_async_copy(
                        rows_v.at[b2], acc_sh.at[dbuf.at[p].at[0]],
                        ssems[b2]).wait()
            elif j == 1:
                @pl.when(g > 0)
                def _():
                    pltpu.make_async_copy(
                        rows_v.at[b2], acc_sh.at[dbuf.at[p].at[0]],
                        ssems[b2]).wait()

                @pl.when(g < ANGRP - 1)
                def _():
                    pltpu.async_copy(
                        src_hbm.at[c, s, pl.ds((g + 1) * AG, AG), :],
                        sbuf.at[1 - p], isem)
                    pltpu.async_copy(
                        dst_hbm.at[c, s, pl.ds((g + 1) * AG, AG), :],
                        dbuf.at[1 - p], isem)
            else:
                pltpu.make_async_copy(
                    rows_v.at[b2], acc_sh.at[dbuf.at[p].at[0]],
                    ssems[b2]).wait()
            if j < AG - 2:
                pltpu.async_copy(table_hbm.at[sbuf.at[p].at[j + 2]],
                                 rows_v.at[b2], gsems[b2])
            else:
                @pl.when(g < ANGRP - 1)
                def _():
                    if j == AG - 2:
                        pltpu.make_async_copy(
                            src_hbm.at[c, s, pl.ds(0, AG), :], sbuf.at[1 - p],
                            isem).wait()
                        pltpu.make_async_copy(
                            dst_hbm.at[c, s, pl.ds(0, AG), :], dbuf.at[1 - p],
                            isem).wait()
                    pltpu.async_copy(
                        table_hbm.at[sbuf.at[1 - p].at[j - (AG - 2)]],
                        rows_v.at[b2], gsems[b2])
        return 0
    lax.fori_loop(0, ANGRP, grp, 0)

    pltpu.make_async_copy(rows_v.at[2],
                          acc_sh.at[dbuf.at[(ANGRP - 1) % 2].at[AG - 1]],
                          ssems[2]).wait()
    pltpu.make_async_copy(rows_v.at[3],
                          acc_sh.at[dbuf.at[(ANGRP - 1) % 2].at[AG - 1]],
                          ssems[3]).wait()

    plsc.subcore_barrier()
    pltpu.sync_copy(acc_sh.at[pl.ds(s * ROWS_PER_TILE, ROWS_PER_TILE), :],
                    out_hbm.at[c, pl.ds(s * ROWS_PER_TILE, ROWS_PER_TILE), :])


_agg_kernel = functools.partial(
    pl.kernel,
    mesh=_mesh,
    out_type=jax.ShapeDtypeStruct((NSC, N_PAD, DIM), jnp.float32),
    scratch_types=[
        pltpu.VMEM((2, AG, ACH), jnp.int32),
        pltpu.VMEM((2, AG, ACH), jnp.int32),
        pltpu.VMEM((NBUF, ACH, DIM), jnp.float32),
        pltpu.SemaphoreType.DMA,
        pltpu.SemaphoreType.DMA,
        pltpu.SemaphoreType.DMA,
        pltpu.SemaphoreType.DMA,
        pltpu.SemaphoreType.DMA,
        pltpu.SemaphoreType.DMA,
        pltpu.SemaphoreType.DMA,
        pltpu.SemaphoreType.DMA,
        pltpu.SemaphoreType.DMA,
        pltpu.VMEM_SHARED((N_PAD, DIM), jnp.float32),
    ],
)(_agg_body)


_BLK = 256
_NBLK = NSC * N_PAD // _BLK
_BLK_PER_G = N_PAD // _BLK


def _row_spec():
    return pl.BlockSpec((_BLK, DIM), lambda i: (i, 0))


def _deg_spec():
    return pl.BlockSpec((_BLK, 1), lambda i: (i, 0))


def _full_spec():
    return pl.BlockSpec((DIM, DIM), lambda i: (0, 0))


def _tc_matmul_scale(x_ref, w_ref, deg_ref, out_ref):
    out_ref[...] = jnp.dot(x_ref[...], w_ref[...],
                           preferred_element_type=jnp.float32) * lax.rsqrt(
                               deg_ref[...] + 1.0)


def _row_mask():
    i = pl.program_id(0)
    g = i // _BLK_PER_G
    base = i * _BLK - g * N_PAD
    rows = base + lax.broadcasted_iota(jnp.int32, (_BLK, 1), 0)
    return rows < N_NODES


def _tc_layer1(s_ref, h_ref, deg_ref, w_ref, b_ref, out_ref):
    inv = lax.rsqrt(deg_ref[...] + 1.0)
    h1 = jnp.maximum((s_ref[...] + h_ref[...]) * inv + b_ref[...], 0.0)
    h1s = jnp.dot(h1, w_ref[...], preferred_element_type=jnp.float32) * inv
    out_ref[...] = jnp.where(_row_mask(), h1s, 0.0)


def _tc_stats(s_ref, h_ref, deg_ref, b_ref, sum_ref, sq_ref):
    i = pl.program_id(0)
    inv = lax.rsqrt(deg_ref[...] + 1.0)
    agg = (s_ref[...] + h_ref[...]) * inv + b_ref[...]
    agg = jnp.where(_row_mask(), agg, 0.0)

    @pl.when(i % _BLK_PER_G == 0)
    def _():
        sum_ref[...] = jnp.zeros_like(sum_ref)
        sq_ref[...] = jnp.zeros_like(sq_ref)

    sum_ref[...] += jnp.sum(agg, axis=0)[None, None, :]
    sq_ref[...] += jnp.sum(agg * agg, axis=0)[None, None, :]


def _tc_standardize(s_ref, h_ref, deg_ref, b_ref, sum_ref, sq_ref, out_ref):
    inv = lax.rsqrt(deg_ref[...] + 1.0)
    agg = (s_ref[...] + h_ref[...]) * inv + b_ref[...]
    n = jnp.float32(N_NODES)
    mean = sum_ref[0] / n
    var = (sq_ref[0] - n * mean * mean) / (n - 1.0)
    out_ref[...] = (agg - mean) * lax.rsqrt(var)


def kernel(x1, edge_index1, x2, edge_index2, W0, b0, W1, b1):
    f32 = jnp.float32
    pad_n = N_PAD - N_NODES
    x_cat = jnp.concatenate([
        x1, jnp.zeros((pad_n, DIM), f32),
        x2, jnp.zeros((pad_n, DIM), f32)], axis=0)

    pad_e = E_PAD - N_EDGES
    def prep(ei, g):
        src = jnp.concatenate(
            [ei[0], jnp.full((pad_e,), SRC_FILL, jnp.int32)]) + g * N_PAD
        dst = jnp.concatenate(
            [ei[1], jnp.full((pad_e,), DST_FILL, jnp.int32)])
        return src, dst
    s1, d1 = prep(edge_index1, 0)
    s2, d2 = prep(edge_index2, 1)
    src_cat = jnp.stack([s1, s2]).reshape(NSC, NTILES, A_CHUNKS, ACH)
    dst_cat = jnp.stack([d1, d2]).reshape(NSC, NTILES, A_CHUNKS, ACH)
    dst_deg = jnp.stack([d1, d2]).reshape(NSC, NTILES, N_CHUNKS, CHUNK)

    deg = _deg_kernel(dst_deg).reshape(NSC * N_PAD, 1)

    b0r = b0.reshape(1, DIM)
    b1r = b1.reshape(1, DIM)
    grid = (_NBLK,)

    h0s = pl.pallas_call(
        _tc_matmul_scale,
        grid=grid,
        in_specs=[_row_spec(), _full_spec(), _deg_spec()],
        out_specs=_row_spec(),
        out_shape=jax.ShapeDtypeStruct((NSC * N_PAD, DIM), f32),
    )(x_cat, W0, deg)

    s0 = _agg_kernel(h0s, src_cat, dst_cat).reshape(NSC * N_PAD, DIM)

    bias_spec = pl.BlockSpec((1, DIM), lambda i: (0, 0))
    h1s = pl.pallas_call(
        _tc_layer1,
        grid=grid,
        in_specs=[_row_spec(), _row_spec(), _deg_spec(), _full_spec(), bias_spec],
        out_specs=_row_spec(),
        out_shape=jax.ShapeDtypeStruct((NSC * N_PAD, DIM), f32),
    )(s0, h0s, deg, W1, b0r)

    s1agg = _agg_kernel(h1s, src_cat, dst_cat).reshape(NSC * N_PAD, DIM)

    stat_spec = pl.BlockSpec((1, 1, DIM), lambda i: (i // _BLK_PER_G, 0, 0))
    colsum, colsq = pl.pallas_call(
        _tc_stats,
        grid=grid,
        in_specs=[_row_spec(), _row_spec(), _deg_spec(), bias_spec],
        out_specs=[stat_spec, stat_spec],
        out_shape=[jax.ShapeDtypeStruct((NSC, 1, DIM), f32)] * 2,
    )(s1agg, h1s, deg, b1r)

    z = pl.pallas_call(
        _tc_standardize,
        grid=grid,
        in_specs=[_row_spec(), _row_spec(), _deg_spec(), bias_spec,
                  stat_spec, stat_spec],
        out_specs=_row_spec(),
        out_shape=jax.ShapeDtypeStruct((NSC * N_PAD, DIM), f32),
    )(s1agg, h1s, deg, b1r, colsum, colsq)

    return z[:N_NODES], z[N_PAD:N_PAD + N_NODES]
